# baseline (speedup 1.0000x reference)
import functools

import jax
import jax.numpy as jnp
from jax import lax
from jax.experimental import pallas as pl
from jax.experimental.pallas import tpu as pltpu
from jax.experimental.pallas import tpu_sc as plsc

_STAGE = 5
N_NODES = 10000
E = 160000
H = 1024
NCH = 4
CH = H // NCH
TE = 1280
TN = 1000

NRANGE = 32
RANGE = 320
N_PAD = NRANGE * RANGE
NSUB = 16
SUBN = RANGE // NSUB
CAP = 6144
GCAP = 448
GP = 112
NPIECE = GCAP // GP
DSTCHUNK = 8000


def _edge_mlp(ea, w1, b1, w2, b2):
    def body(ea_ref, w1_ref, b1_ref, w2_ref, b2_ref, o_ref):
        x = ea_ref[...].astype(jnp.bfloat16)
        h = jnp.dot(x, w1_ref[...], preferred_element_type=jnp.float32)
        h = jnp.maximum(h + b1_ref[...], 0.0).astype(jnp.bfloat16)
        v = jnp.dot(h, w2_ref[...], preferred_element_type=jnp.float32)
        v = (v + b2_ref[...]).astype(jnp.bfloat16)
        for c in range(NCH):
            o_ref[c] = v[:, c * CH:(c + 1) * CH]

    return pl.pallas_call(
        body,
        grid=(E // TE,),
        in_specs=[
            pl.BlockSpec((TE, 16), lambda i: (i, 0)),
            pl.BlockSpec((16, H), lambda i: (0, 0)),
            pl.BlockSpec((1, H), lambda i: (0, 0)),
            pl.BlockSpec((H, H), lambda i: (0, 0)),
            pl.BlockSpec((1, H), lambda i: (0, 0)),
        ],
        out_specs=pl.BlockSpec((NCH, TE, CH), lambda i: (0, i, 0)),
        out_shape=jax.ShapeDtypeStruct((NCH, E, CH), jnp.bfloat16),
    )(ea, w1, b1, w2, b2)


def _sc_segment_max(vecflat, dst):
    mesh = plsc.VectorSubcoreMesh(core_axis_name="c", subcore_axis_name="s")

    @functools.partial(
        pl.kernel,
        out_type=jax.ShapeDtypeStruct((N_PAD, H // 2), jnp.int32),
        mesh=mesh,
        compiler_params=pltpu.CompilerParams(needs_layout_passes=False),
        scratch_types=[
            pltpu.VMEM((DSTCHUNK,), jnp.int32),
            pltpu.VMEM((CAP + 16,), jnp.int32),
            pltpu.VMEM((CAP + 16,), jnp.int32),
            pltpu.VMEM((CAP + GCAP + 128,), jnp.int32),
            pltpu.VMEM((GCAP,), jnp.int32),
            pltpu.VMEM((RANGE + 16,), jnp.int32),
            pltpu.VMEM((RANGE + 16,), jnp.int32),
            pltpu.VMEM((RANGE + 16,), jnp.int32),
            pltpu.VMEM((RANGE, CH // 2), jnp.int32),
            pltpu.VMEM((GCAP, CH // 2), jnp.int32),
            pltpu.SemaphoreType.DMA,
        ],
    )
    def k(vec_hbm, dst_hbm, out_hbm, dstchunk, idbuf, lnbuf, idbuf2, idxg,
          counts, offs, cur, acc, rowbuf, sem):
        wid = lax.axis_index("s") * 2 + lax.axis_index("c")
        lo = wid * RANGE
        hi = lo + RANGE
        iota = lax.iota(jnp.int32, 16)
        zi16 = jnp.zeros((16,), jnp.int32)
        lane0 = iota == 0

        def sget(ref, i):
            return ref[pl.ds(i, 16)][0]

        def sput(ref, i, v):
            plsc.store_scatter(ref, [jnp.full((16,), i, jnp.int32)],
                               jnp.full((16,), v, jnp.int32), mask=lane0)

        def z1(t, _):
            idbuf2[pl.ds(16 * t, 16)] = zi16
            return 0
        lax.fori_loop(0, (CAP + GCAP + 128) // 16, z1, 0)

        def z2(t, _):
            counts[pl.ds(16 * t, 16)] = zi16
            return 0
        lax.fori_loop(0, RANGE // 16, z2, 0)

        def fchunk(kc, off):
            pltpu.sync_copy(dst_hbm.at[pl.ds(kc * DSTCHUNK, DSTCHUNK)],
                            dstchunk)

            def fvec(t, off):
                d = dstchunk[pl.ds(16 * t, 16)]
                ids = kc * DSTCHUNK + 16 * t + iota
                m = (d >= lo) & (d < hi)
                csum = plsc.cumsum(m.astype(jnp.int32))
                posn = off + csum - 1
                plsc.store_scatter(lnbuf, [posn], d - lo, mask=m)
                plsc.store_scatter(idbuf, [posn], ids, mask=m)
                return jnp.minimum(off + csum[15], CAP)

            return lax.fori_loop(0, DSTCHUNK // 16, fvec, off)

        off = lax.fori_loop(0, E // DSTCHUNK, fchunk, jnp.int32(0))
        nvec = (off + 15) // 16

        if _STAGE >= 2:
         def hist(t, _):
            ln = lnbuf[pl.ds(16 * t, 16)]
            m = (16 * t + iota) < off
            plsc.addupdate_scatter(counts, [ln],
                                   jnp.ones((16,), jnp.int32), mask=m)
            return 0
         lax.fori_loop(0, nvec, hist, 0)

        if _STAGE >= 3:
         def pfx(t, carry):
            v = counts[pl.ds(16 * t, 16)]
            inc = plsc.cumsum(v)
            exc = inc - v + carry
            offs[pl.ds(16 * t, 16)] = exc
            cur[pl.ds(16 * t, 16)] = exc
            return carry + jnp.max(inc)
         lax.fori_loop(0, RANGE // 16, pfx, jnp.int32(0))

        if _STAGE >= 4:
         def scat(i, _):
            ln = sget(lnbuf, i)
            p = sget(cur, ln)
            sput(idbuf2, p, sget(idbuf, i))
            sput(cur, ln, p + 1)
            return 0
         lax.fori_loop(0, off, scat, 0)

        zb = jnp.zeros((32,), jnp.bfloat16)

        if _STAGE >= 5:
         def chunk_body(c, _):
            coff = c * E

            def sub_body(s, _):
                g0 = sget(offs, s * SUBN)

                def bidx(t, _):
                    idxg[pl.ds(16 * t, 16)] = (
                        idbuf2[pl.ds(g0 + 16 * t, 16)] + coff)
                    return 0
                lax.fori_loop(0, GCAP // 16, bidx, 0)

                copies = [
                    pltpu.async_copy(
                        vec_hbm.at[idxg.at[pl.ds(GP * p_, GP)]],
                        rowbuf.at[pl.ds(GP * p_, GP), :], sem)
                    for p_ in range(NPIECE)
                ]
                for cp in copies:
                    cp.wait()

                def node_body(jn, _):
                    ln = s * SUBN + jn
                    cnt = sget(counts, ln)
                    p = sget(offs, ln) - g0
                    pc = jnp.minimum(p, GCAP - 1)
                    ec = jnp.minimum(p + cnt, GCAP)

                    def nonempty():
                        ms = tuple(
                            plsc.bitcast(rowbuf[pc, pl.ds(16 * q, 16)],
                                         jnp.bfloat16)
                            for q in range(CH // 32))

                        def rmax(j, ms):
                            return tuple(
                                jnp.maximum(
                                    ms[q],
                                    plsc.bitcast(
                                        rowbuf[j, pl.ds(16 * q, 16)],
                                        jnp.bfloat16))
                                for q in range(CH // 32))
                        return lax.fori_loop(pc + 1, ec, rmax, ms)

                    def empty():
                        return tuple(zb for _ in range(CH // 32))

                    ms = lax.cond(cnt > 0, nonempty, empty)
                    for q in range(CH // 32):
                        acc[ln, pl.ds(16 * q, 16)] = plsc.bitcast(
                            ms[q], jnp.int32)
                    return 0
                lax.fori_loop(0, SUBN, node_body, 0)
                return 0
            lax.fori_loop(0, NSUB, sub_body, 0)

            pltpu.sync_copy(
                acc,
                out_hbm.at[pl.ds(lo, RANGE), pl.ds(c * (CH // 2), CH // 2)])
            return 0
         lax.fori_loop(0, NCH, chunk_body, 0)

    return k(vecflat, dst)


def _node_mlps(nv, action, wa1, ba1, wa2, ba2, wf1, bf1, wf2row):
    def body(nv_ref, act_ref, wa1_ref, ba1_ref, wa2_ref, ba2_ref, wf1_ref,
             bf1_ref, wf2_ref, o_ref):
        a = act_ref[...].astype(jnp.bfloat16)
        t = jnp.dot(a, wa1_ref[...], preferred_element_type=jnp.float32)
        t = jnp.maximum(t + ba1_ref[...], 0.0).astype(jnp.bfloat16)
        emb = jnp.dot(t, wa2_ref[...], preferred_element_type=jnp.float32)
        emb = emb + ba2_ref[...]
        feat = (nv_ref[...].astype(jnp.float32) + emb).astype(jnp.bfloat16)
        h = jnp.dot(feat, wf1_ref[...], preferred_element_type=jnp.float32)
        h = jnp.maximum(h + bf1_ref[...], 0.0)
        fld = jnp.sum(h * wf2_ref[...].astype(jnp.float32), axis=1)
        o_ref[pl.program_id(0), :] = fld

    return pl.pallas_call(
        body,
        grid=(N_NODES // TN,),
        in_specs=[
            pl.BlockSpec((TN, H), lambda i: (i, 0)),
            pl.BlockSpec((TN, 16), lambda i: (i, 0)),
            pl.BlockSpec((16, H), lambda i: (0, 0)),
            pl.BlockSpec((1, H), lambda i: (0, 0)),
            pl.BlockSpec((H, H), lambda i: (0, 0)),
            pl.BlockSpec((1, H), lambda i: (0, 0)),
            pl.BlockSpec((H, H), lambda i: (0, 0)),
            pl.BlockSpec((1, H), lambda i: (0, 0)),
            pl.BlockSpec((1, H), lambda i: (0, 0)),
        ],
        out_specs=pl.BlockSpec((N_NODES // TN, TN), lambda i: (0, 0)),
        out_shape=jax.ShapeDtypeStruct((N_NODES // TN, TN), jnp.float32),
    )(nv, action, wa1, ba1, wa2, ba2, wf1, bf1, wf2row)


def kernel(edge_attr, edge_index, action, Wv1, bv1, Wv2, bv2, Wa1, ba1,
           Wa2, ba2, Wf1, bf1, Wf2, bf2):
    dst = edge_index[1].astype(jnp.int32)
    bf16 = jnp.bfloat16
    vecc = _edge_mlp(edge_attr, Wv1.astype(bf16), bv1.reshape(1, H),
                     Wv2.astype(bf16), bv2.reshape(1, H))
    vec_i32 = lax.bitcast_convert_type(
        vecc.reshape(NCH * E, CH // 2, 2), jnp.int32)
    nv_i32 = _sc_segment_max(vec_i32, dst)
    nv = lax.bitcast_convert_type(nv_i32, bf16).reshape(N_PAD, H)
    fld = _node_mlps(nv[:N_NODES], action, Wa1.astype(bf16),
                     ba1.reshape(1, H), Wa2.astype(bf16), ba2.reshape(1, H),
                     Wf1.astype(bf16), bf1.reshape(1, H),
                     Wf2.reshape(1, H).astype(bf16))
    return fld.reshape(N_NODES) + bf2[0]


# speedup vs baseline: 2.6100x; 2.6100x over previous
import functools

import jax
import jax.numpy as jnp
from jax import lax
from jax.experimental import pallas as pl
from jax.experimental.pallas import tpu as pltpu
from jax.experimental.pallas import tpu_sc as plsc

_STAGE = 5
N_NODES = 10000
E = 160000
H = 1024
NCH = 4
CH = H // NCH
TE = 1280
TN = 1000

NRANGE = 32
RANGE = 320
N_PAD = NRANGE * RANGE
NSUB = 16
SUBN = RANGE // NSUB
CAP = 6144
GCAP = 448
GP = 112
NPIECE = GCAP // GP
DSTCHUNK = 8000


def _edge_mlp(ea, w1, b1, w2, b2):
    def body(ea_ref, w1_ref, b1_ref, w2_ref, b2_ref, o_ref):
        x = ea_ref[...].astype(jnp.bfloat16)
        h = jnp.dot(x, w1_ref[...], preferred_element_type=jnp.float32)
        h = jnp.maximum(h + b1_ref[...], 0.0).astype(jnp.bfloat16)
        v = jnp.dot(h, w2_ref[...], preferred_element_type=jnp.float32)
        v = (v + b2_ref[...]).astype(jnp.bfloat16)
        lo = lax.bitcast_convert_type(v[:, :H // 2], jnp.uint16)
        hi = lax.bitcast_convert_type(v[:, H // 2:], jnp.uint16)
        packed = lo.astype(jnp.uint32) | (hi.astype(jnp.uint32) << 16)
        packed = lax.bitcast_convert_type(packed, jnp.int32)
        for c in range(NCH):
            o_ref[c] = packed[:, c * (CH // 2):(c + 1) * (CH // 2)]

    return pl.pallas_call(
        body,
        grid=(E // TE,),
        in_specs=[
            pl.BlockSpec((TE, 16), lambda i: (i, 0)),
            pl.BlockSpec((16, H), lambda i: (0, 0)),
            pl.BlockSpec((1, H), lambda i: (0, 0)),
            pl.BlockSpec((H, H), lambda i: (0, 0)),
            pl.BlockSpec((1, H), lambda i: (0, 0)),
        ],
        out_specs=pl.BlockSpec((NCH, TE, CH // 2), lambda i: (0, i, 0)),
        out_shape=jax.ShapeDtypeStruct((NCH, E, CH // 2), jnp.int32),
    )(ea, w1, b1, w2, b2)


def _sc_segment_max(vecflat, dst):
    mesh = plsc.VectorSubcoreMesh(core_axis_name="c", subcore_axis_name="s")

    @functools.partial(
        pl.kernel,
        out_type=jax.ShapeDtypeStruct((N_PAD, H // 2), jnp.int32),
        mesh=mesh,
        compiler_params=pltpu.CompilerParams(needs_layout_passes=False),
        scratch_types=[
            pltpu.VMEM((DSTCHUNK,), jnp.int32),
            pltpu.VMEM((CAP + 16,), jnp.int32),
            pltpu.VMEM((CAP + 16,), jnp.int32),
            pltpu.VMEM((CAP + GCAP + 128,), jnp.int32),
            pltpu.VMEM((GCAP,), jnp.int32),
            pltpu.VMEM((RANGE + 16,), jnp.int32),
            pltpu.VMEM((RANGE + 16,), jnp.int32),
            pltpu.VMEM((RANGE + 16,), jnp.int32),
            pltpu.VMEM((RANGE, CH // 2), jnp.int32),
            pltpu.VMEM((GCAP, CH // 2), jnp.int32),
            pltpu.SemaphoreType.DMA,
        ],
    )
    def k(vec_hbm, dst_hbm, out_hbm, dstchunk, idbuf, lnbuf, idbuf2, idxg,
          counts, offs, cur, acc, rowbuf, sem):
        wid = lax.axis_index("s") * 2 + lax.axis_index("c")
        lo = wid * RANGE
        hi = lo + RANGE
        iota = lax.iota(jnp.int32, 16)
        zi16 = jnp.zeros((16,), jnp.int32)
        lane0 = iota == 0

        def sget(ref, i):
            return ref[pl.ds(i, 16)][0]

        def sput(ref, i, v):
            plsc.store_scatter(ref, [jnp.full((16,), i, jnp.int32)],
                               jnp.full((16,), v, jnp.int32), mask=lane0)

        def z1(t, _):
            idbuf2[pl.ds(16 * t, 16)] = zi16
            return 0
        lax.fori_loop(0, (CAP + GCAP + 128) // 16, z1, 0)

        def z2(t, _):
            counts[pl.ds(16 * t, 16)] = zi16
            return 0
        lax.fori_loop(0, RANGE // 16, z2, 0)

        def fchunk(kc, off):
            pltpu.sync_copy(dst_hbm.at[pl.ds(kc * DSTCHUNK, DSTCHUNK)],
                            dstchunk)

            def fvec(t, off):
                d = dstchunk[pl.ds(16 * t, 16)]
                ids = kc * DSTCHUNK + 16 * t + iota
                m = (d >= lo) & (d < hi)
                csum = plsc.cumsum(m.astype(jnp.int32))
                posn = off + csum - 1
                plsc.store_scatter(lnbuf, [posn], d - lo, mask=m)
                plsc.store_scatter(idbuf, [posn], ids, mask=m)
                return jnp.minimum(off + csum[15], CAP)

            return lax.fori_loop(0, DSTCHUNK // 16, fvec, off)

        off = lax.fori_loop(0, E // DSTCHUNK, fchunk, jnp.int32(0))
        nvec = (off + 15) // 16

        if _STAGE >= 2:
         def hist(t, _):
            ln = lnbuf[pl.ds(16 * t, 16)]
            m = (16 * t + iota) < off
            plsc.addupdate_scatter(counts, [ln],
                                   jnp.ones((16,), jnp.int32), mask=m)
            return 0
         lax.fori_loop(0, nvec, hist, 0)

        if _STAGE >= 3:
         def pfx(t, carry):
            v = counts[pl.ds(16 * t, 16)]
            inc = plsc.cumsum(v)
            exc = inc - v + carry
            offs[pl.ds(16 * t, 16)] = exc
            cur[pl.ds(16 * t, 16)] = exc
            return carry + jnp.max(inc)
         lax.fori_loop(0, RANGE // 16, pfx, jnp.int32(0))

        if _STAGE >= 4:
         SENT = jnp.int32(0x7FFFFFF)
         ones16 = jnp.ones((16,), jnp.int32)

         def scat(t, _):
            base = 16 * t
            valid = (base + iota) < off
            lns = jnp.where(valid, lnbuf[pl.ds(base, 16)], SENT)
            idv = idbuf[pl.ds(base, 16)]
            sk, sv = plsc.sort_key_val(lns, iota)
            prev = sk.at[jnp.maximum(iota - 1, 0)].get(
                mode="promise_in_bounds")
            runstart = (sk != prev) | (iota == 0)
            firstpos = plsc.cummax(jnp.where(runstart, iota, 0))
            rank = iota - firstpos
            skc = jnp.minimum(sk, RANGE - 1)
            pos = plsc.load_gather(cur, [skc]) + rank
            ids_sorted = idv.at[sv].get(mode="promise_in_bounds")
            validm = sk != SENT
            plsc.store_scatter(idbuf2, [pos], ids_sorted, mask=validm)
            plsc.addupdate_scatter(cur, [skc], ones16, mask=validm)
            return 0
         lax.fori_loop(0, nvec, scat, 0)

        zb = jnp.zeros((32,), jnp.bfloat16)

        if _STAGE >= 5:
         def chunk_body(c, _):
            coff = c * E

            def sub_body(s, _):
                g0 = sget(offs, s * SUBN)

                def bidx(t, _):
                    idxg[pl.ds(16 * t, 16)] = (
                        idbuf2[pl.ds(g0 + 16 * t, 16)] + coff)
                    return 0
                lax.fori_loop(0, GCAP // 16, bidx, 0)

                copies = [
                    pltpu.async_copy(
                        vec_hbm.at[idxg.at[pl.ds(GP * p_, GP)]],
                        rowbuf.at[pl.ds(GP * p_, GP), :], sem)
                    for p_ in range(NPIECE)
                ]
                for cp in copies:
                    cp.wait()

                def node_body(jn, _):
                    ln = s * SUBN + jn
                    cnt = sget(counts, ln)
                    p = sget(offs, ln) - g0
                    pc = jnp.minimum(p, GCAP - 1)
                    ec = jnp.minimum(p + cnt, GCAP)

                    def nonempty():
                        ms = tuple(
                            plsc.bitcast(rowbuf[pc, pl.ds(16 * q, 16)],
                                         jnp.bfloat16)
                            for q in range(CH // 32))

                        def rmax(j, ms):
                            return tuple(
                                jnp.maximum(
                                    ms[q],
                                    plsc.bitcast(
                                        rowbuf[j, pl.ds(16 * q, 16)],
                                        jnp.bfloat16))
                                for q in range(CH // 32))
                        return lax.fori_loop(pc + 1, ec, rmax, ms)

                    def empty():
                        return tuple(zb for _ in range(CH // 32))

                    ms = lax.cond(cnt > 0, nonempty, empty)
                    for q in range(CH // 32):
                        acc[ln, pl.ds(16 * q, 16)] = plsc.bitcast(
                            ms[q], jnp.int32)
                    return 0
                lax.fori_loop(0, SUBN, node_body, 0)
                return 0
            lax.fori_loop(0, NSUB, sub_body, 0)

            pltpu.sync_copy(
                acc,
                out_hbm.at[pl.ds(lo, RANGE), pl.ds(c * (CH // 2), CH // 2)])
            return 0
         lax.fori_loop(0, NCH, chunk_body, 0)

    return k(vecflat, dst)


def _node_mlps(nv, action, wa1, ba1, wa2, ba2, wf1, bf1, wf2row):
    def body(nv_ref, act_ref, wa1_ref, ba1_ref, wa2_ref, ba2_ref, wf1_ref,
             bf1_ref, wf2_ref, o_ref):
        a = act_ref[...].astype(jnp.bfloat16)
        t = jnp.dot(a, wa1_ref[...], preferred_element_type=jnp.float32)
        t = jnp.maximum(t + ba1_ref[...], 0.0).astype(jnp.bfloat16)
        emb = jnp.dot(t, wa2_ref[...], preferred_element_type=jnp.float32)
        emb = emb + ba2_ref[...]
        u = lax.bitcast_convert_type(nv_ref[...], jnp.uint32)
        nvlo = lax.bitcast_convert_type(
            (u & 0xFFFF).astype(jnp.uint16), jnp.bfloat16)
        nvhi = lax.bitcast_convert_type(
            (u >> 16).astype(jnp.uint16), jnp.bfloat16)
        nv = jnp.concatenate([nvlo, nvhi], axis=1)
        feat = (nv.astype(jnp.float32) + emb).astype(jnp.bfloat16)
        h = jnp.dot(feat, wf1_ref[...], preferred_element_type=jnp.float32)
        h = jnp.maximum(h + bf1_ref[...], 0.0)
        fld = jnp.sum(h * wf2_ref[...].astype(jnp.float32), axis=1)
        o_ref[pl.program_id(0), :] = fld

    return pl.pallas_call(
        body,
        grid=(N_NODES // TN,),
        in_specs=[
            pl.BlockSpec((TN, H // 2), lambda i: (i, 0)),
            pl.BlockSpec((TN, 16), lambda i: (i, 0)),
            pl.BlockSpec((16, H), lambda i: (0, 0)),
            pl.BlockSpec((1, H), lambda i: (0, 0)),
            pl.BlockSpec((H, H), lambda i: (0, 0)),
            pl.BlockSpec((1, H), lambda i: (0, 0)),
            pl.BlockSpec((H, H), lambda i: (0, 0)),
            pl.BlockSpec((1, H), lambda i: (0, 0)),
            pl.BlockSpec((1, H), lambda i: (0, 0)),
        ],
        out_specs=pl.BlockSpec((N_NODES // TN, TN), lambda i: (0, 0)),
        out_shape=jax.ShapeDtypeStruct((N_NODES // TN, TN), jnp.float32),
    )(nv, action, wa1, ba1, wa2, ba2, wf1, bf1, wf2row)


def kernel(edge_attr, edge_index, action, Wv1, bv1, Wv2, bv2, Wa1, ba1,
           Wa2, ba2, Wf1, bf1, Wf2, bf2):
    dst = edge_index[1].astype(jnp.int32)
    bf16 = jnp.bfloat16
    vecc = _edge_mlp(edge_attr, Wv1.astype(bf16), bv1.reshape(1, H),
                     Wv2.astype(bf16), bv2.reshape(1, H))
    nv_i32 = _sc_segment_max(vecc.reshape(NCH * E, CH // 2), dst)
    fld = _node_mlps(nv_i32, action, Wa1.astype(bf16),
                     ba1.reshape(1, H), Wa2.astype(bf16), ba2.reshape(1, H),
                     Wf1.astype(bf16), bf1.reshape(1, H),
                     Wf2.reshape(1, H).astype(bf16))
    return fld.reshape(N_NODES) + bf2[0]


# speedup vs baseline: 3.8733x; 1.4840x over previous
import functools

import jax
import jax.numpy as jnp
from jax import lax
from jax.experimental import pallas as pl
from jax.experimental.pallas import tpu as pltpu
from jax.experimental.pallas import tpu_sc as plsc

N_NODES = 10000
E = 160000
H = 1024
NCH = 4
CW = 128
TE = 1280
TN = 1000

NRANGE = 32
RANGE = 320
N_PAD = NRANGE * RANGE
SUBN = 10
NSUB = RANGE // SUBN
CAP = 5888
GCAP = 256
GP = 128
DSTCHUNK = 4000


def _edge_mlp(ea, w1, b1, w2, b2):
    def body(ea_ref, w1_ref, b1_ref, w2_ref, b2_ref, o_ref):
        x = ea_ref[...].astype(jnp.bfloat16)
        h = jnp.dot(x, w1_ref[...], preferred_element_type=jnp.float32)
        h = jnp.maximum(h + b1_ref[...], 0.0).astype(jnp.bfloat16)
        v = jnp.dot(h, w2_ref[...], preferred_element_type=jnp.float32)
        v = (v + b2_ref[...]).astype(jnp.bfloat16)
        lo = lax.bitcast_convert_type(v[:, :H // 2], jnp.uint16)
        hi = lax.bitcast_convert_type(v[:, H // 2:], jnp.uint16)
        packed = lo.astype(jnp.uint32) | (hi.astype(jnp.uint32) << 16)
        packed = lax.bitcast_convert_type(packed, jnp.int32)
        for c in range(NCH):
            o_ref[c] = packed[:, c * CW:(c + 1) * CW]

    return pl.pallas_call(
        body,
        grid=(E // TE,),
        in_specs=[
            pl.BlockSpec((TE, 16), lambda i: (i, 0)),
            pl.BlockSpec((16, H), lambda i: (0, 0)),
            pl.BlockSpec((1, H), lambda i: (0, 0)),
            pl.BlockSpec((H, H), lambda i: (0, 0)),
            pl.BlockSpec((1, H), lambda i: (0, 0)),
        ],
        out_specs=pl.BlockSpec((NCH, TE, CW), lambda i: (0, i, 0)),
        out_shape=jax.ShapeDtypeStruct((NCH, E, CW), jnp.int32),
    )(ea, w1, b1, w2, b2)


def _sc_segment_max(vecflat, dst):
    mesh = plsc.VectorSubcoreMesh(core_axis_name="c", subcore_axis_name="s")

    @functools.partial(
        pl.kernel,
        out_type=jax.ShapeDtypeStruct((N_PAD, NCH * CW), jnp.int32),
        mesh=mesh,
        compiler_params=pltpu.CompilerParams(needs_layout_passes=False),
        scratch_types=[
            pltpu.VMEM((DSTCHUNK,), jnp.int32),
            pltpu.VMEM((CAP + 16,), jnp.int32),
            pltpu.VMEM((CAP + 16,), jnp.int32),
            pltpu.VMEM((CAP + GCAP + 128,), jnp.int32),
            pltpu.VMEM((GCAP,), jnp.int32),
            pltpu.VMEM((GCAP,), jnp.int32),
            pltpu.VMEM((RANGE + 16,), jnp.int32),
            pltpu.VMEM((RANGE + 16,), jnp.int32),
            pltpu.VMEM((RANGE + 16,), jnp.int32),
            pltpu.VMEM((RANGE, CW), jnp.int32),
            pltpu.VMEM((GCAP, CW), jnp.int32),
            pltpu.VMEM((GCAP, CW), jnp.int32),
            pltpu.SemaphoreType.DMA,
            pltpu.SemaphoreType.DMA,
        ],
    )
    def k(vec_hbm, dst_hbm, out_hbm, dstchunk, idbuf, lnbuf, idbuf2,
          idxga, idxgb, counts, offs, cur, acc, rowbufa, rowbufb,
          sema, semb):
        wid = lax.axis_index("s") * 2 + lax.axis_index("c")
        lo = wid * RANGE
        hi = lo + RANGE
        iota = lax.iota(jnp.int32, 16)
        zi16 = jnp.zeros((16,), jnp.int32)
        lane0 = iota == 0

        def sget(ref, i):
            return ref[pl.ds(i, 16)][0]

        def sput(ref, i, v):
            plsc.store_scatter(ref, [jnp.full((16,), i, jnp.int32)],
                               jnp.full((16,), v, jnp.int32), mask=lane0)

        def z1(t, _):
            idbuf2[pl.ds(16 * t, 16)] = zi16
            return 0
        lax.fori_loop(0, (CAP + GCAP + 128) // 16, z1, 0)

        def z2(t, _):
            counts[pl.ds(16 * t, 16)] = zi16
            return 0
        lax.fori_loop(0, RANGE // 16, z2, 0)

        def fchunk(kc, off):
            pltpu.sync_copy(dst_hbm.at[pl.ds(kc * DSTCHUNK, DSTCHUNK)],
                            dstchunk)

            def fvec(t, off):
                d = dstchunk[pl.ds(16 * t, 16)]
                ids = kc * DSTCHUNK + 16 * t + iota
                m = (d >= lo) & (d < hi)
                csum = plsc.cumsum(jnp.where(m, 1, 0))
                posn = off + csum - 1
                plsc.store_scatter(lnbuf, [posn], d - lo, mask=m)
                plsc.store_scatter(idbuf, [posn], ids, mask=m)
                return jnp.minimum(off + csum[15], CAP)

            return lax.fori_loop(0, DSTCHUNK // 16, fvec, off)

        off = lax.fori_loop(0, E // DSTCHUNK, fchunk, jnp.int32(0))
        nvec = (off + 15) // 16

        ones16 = jnp.ones((16,), jnp.int32)

        def hist(t, _):
            ln = lnbuf[pl.ds(16 * t, 16)]
            m = (16 * t + iota) < off
            plsc.addupdate_scatter(counts, [ln], ones16, mask=m)
            return 0
        lax.fori_loop(0, nvec, hist, 0)

        def pfx(t, carry):
            v = counts[pl.ds(16 * t, 16)]
            inc = plsc.cumsum(v)
            exc = inc - v + carry
            offs[pl.ds(16 * t, 16)] = exc
            cur[pl.ds(16 * t, 16)] = exc
            return carry + jnp.max(inc)
        lax.fori_loop(0, RANGE // 16, pfx, jnp.int32(0))
        sput(offs, jnp.int32(RANGE), off)

        SENT = jnp.int32(0x7FFFFFF)

        def scat(t, _):
            base = 16 * t
            valid = (base + iota) < off
            lns = jnp.where(valid, lnbuf[pl.ds(base, 16)], SENT)
            idv = idbuf[pl.ds(base, 16)]
            sk, sv = plsc.sort_key_val(lns, iota)
            prev = sk.at[jnp.maximum(iota - 1, 0)].get(
                mode="promise_in_bounds")
            runstart = (sk != prev) | (iota == 0)
            firstpos = plsc.cummax(jnp.where(runstart, iota, 0))
            rank = iota - firstpos
            skc = jnp.minimum(sk, RANGE - 1)
            pos = plsc.load_gather(cur, [skc]) + rank
            ids_sorted = idv.at[sv].get(mode="promise_in_bounds")
            validm = sk != SENT
            plsc.store_scatter(idbuf2, [pos], ids_sorted, mask=validm)
            plsc.addupdate_scatter(cur, [skc], ones16, mask=validm)
            return 0
        lax.fori_loop(0, nvec, scat, 0)

        NEG = jnp.full((32,), -jnp.inf, jnp.bfloat16)
        zb = jnp.zeros((32,), jnp.bfloat16)

        def npieces(s):
            g0 = sget(offs, s * SUBN)
            nrows = sget(offs, s * SUBN + SUBN) - g0
            return g0, (jnp.minimum(nrows, GCAP) + GP - 1) // GP

        def chunk_body(c, _):
            coff = c * E

            def issue(s, idxg, rowbuf, sem):
                g0, npc = npieces(s)

                def bidx(t, _):
                    idxg[pl.ds(16 * t, 16)] = (
                        idbuf2[pl.ds(g0 + 16 * t, 16)] + coff)
                    return 0
                lax.fori_loop(0, GCAP // 16, bidx, 0)

                def ip(p_, _):
                    pltpu.async_copy(
                        vec_hbm.at[idxg.at[pl.ds(GP * p_, GP)]],
                        rowbuf.at[pl.ds(GP * p_, GP), :], sem)
                    return 0
                lax.fori_loop(0, npc, ip, 0)

            def drain(s, rowbuf, sem):
                _, npc = npieces(s)

                def dp(p_, _):
                    pltpu.make_async_copy(
                        vec_hbm.at[pl.ds(0, GP), :],
                        rowbuf.at[pl.ds(0, GP), :], sem).wait()
                    return 0
                lax.fori_loop(0, npc, dp, 0)

            def reduce(s, rowbuf):
                g0 = sget(offs, s * SUBN)

                def node_body(jn, _):
                    ln = s * SUBN + jn
                    cnt = sget(counts, ln)
                    p = sget(offs, ln) - g0
                    pc = jnp.minimum(p, GCAP - 1)
                    ec = jnp.minimum(p + cnt, GCAP)

                    def rmax(j, ms):
                        return tuple(
                            jnp.maximum(
                                ms[q],
                                plsc.bitcast(rowbuf[j, pl.ds(16 * q, 16)],
                                             jnp.bfloat16))
                            for q in range(CW // 16))
                    ms = lax.fori_loop(pc, ec, rmax,
                                       tuple(NEG for _ in range(CW // 16)))
                    nonz = cnt > 0
                    for q in range(CW // 16):
                        acc[ln, pl.ds(16 * q, 16)] = plsc.bitcast(
                            jnp.where(nonz, ms[q], zb), jnp.int32)
                    return 0
                lax.fori_loop(0, SUBN, node_body, 0)

            issue(0, idxga, rowbufa, sema)

            def pair_body(sp, _):
                s0 = 2 * sp
                issue(s0 + 1, idxgb, rowbufb, semb)
                drain(s0, rowbufa, sema)
                reduce(s0, rowbufa)

                @pl.when(s0 + 2 < NSUB)
                def _():
                    issue(s0 + 2, idxga, rowbufa, sema)
                drain(s0 + 1, rowbufb, semb)
                reduce(s0 + 1, rowbufb)
                return 0
            lax.fori_loop(0, NSUB // 2, pair_body, 0)

            pltpu.sync_copy(acc,
                            out_hbm.at[pl.ds(lo, RANGE), pl.ds(c * CW, CW)])
            return 0
        lax.fori_loop(0, NCH, chunk_body, 0)

    return k(vecflat, dst)


def _node_mlps(nv, action, wa1, ba1, wa2, ba2, wf1, bf1, wf2row):
    def body(nv_ref, act_ref, wa1_ref, ba1_ref, wa2_ref, ba2_ref, wf1_ref,
             bf1_ref, wf2_ref, o_ref):
        a = act_ref[...].astype(jnp.bfloat16)
        t = jnp.dot(a, wa1_ref[...], preferred_element_type=jnp.float32)
        t = jnp.maximum(t + ba1_ref[...], 0.0).astype(jnp.bfloat16)
        emb = jnp.dot(t, wa2_ref[...], preferred_element_type=jnp.float32)
        emb = emb + ba2_ref[...]
        u = lax.bitcast_convert_type(nv_ref[...], jnp.uint32)
        nvlo = lax.bitcast_convert_type(
            (u & 0xFFFF).astype(jnp.uint16), jnp.bfloat16)
        nvhi = lax.bitcast_convert_type(
            (u >> 16).astype(jnp.uint16), jnp.bfloat16)
        nvv = jnp.concatenate([nvlo, nvhi], axis=1)
        feat = (nvv.astype(jnp.float32) + emb).astype(jnp.bfloat16)
        h = jnp.dot(feat, wf1_ref[...], preferred_element_type=jnp.float32)
        h = jnp.maximum(h + bf1_ref[...], 0.0)
        fld = jnp.sum(h * wf2_ref[...].astype(jnp.float32), axis=1)
        o_ref[pl.program_id(0), :] = fld

    return pl.pallas_call(
        body,
        grid=(N_NODES // TN,),
        in_specs=[
            pl.BlockSpec((TN, H // 2), lambda i: (i, 0)),
            pl.BlockSpec((TN, 16), lambda i: (i, 0)),
            pl.BlockSpec((16, H), lambda i: (0, 0)),
            pl.BlockSpec((1, H), lambda i: (0, 0)),
            pl.BlockSpec((H, H), lambda i: (0, 0)),
            pl.BlockSpec((1, H), lambda i: (0, 0)),
            pl.BlockSpec((H, H), lambda i: (0, 0)),
            pl.BlockSpec((1, H), lambda i: (0, 0)),
            pl.BlockSpec((1, H), lambda i: (0, 0)),
        ],
        out_specs=pl.BlockSpec((N_NODES // TN, TN), lambda i: (0, 0)),
        out_shape=jax.ShapeDtypeStruct((N_NODES // TN, TN), jnp.float32),
    )(nv, action, wa1, ba1, wa2, ba2, wf1, bf1, wf2row)


def kernel(edge_attr, edge_index, action, Wv1, bv1, Wv2, bv2, Wa1, ba1,
           Wa2, ba2, Wf1, bf1, Wf2, bf2):
    dst = edge_index[1].astype(jnp.int32)
    bf16 = jnp.bfloat16
    vecc = _edge_mlp(edge_attr, Wv1.astype(bf16), bv1.reshape(1, H),
                     Wv2.astype(bf16), bv2.reshape(1, H))
    nv_i32 = _sc_segment_max(vecc.reshape(NCH * E, CW), dst)
    fld = _node_mlps(nv_i32, action, Wa1.astype(bf16),
                     ba1.reshape(1, H), Wa2.astype(bf16), ba2.reshape(1, H),
                     Wf1.astype(bf16), bf1.reshape(1, H),
                     Wf2.reshape(1, H).astype(bf16))
    return fld.reshape(N_NODES) + bf2[0]


# speedup vs baseline: 4.6945x; 1.2120x over previous
import functools

import jax
import jax.numpy as jnp
from jax import lax
from jax.experimental import pallas as pl
from jax.experimental.pallas import tpu as pltpu
from jax.experimental.pallas import tpu_sc as plsc

N_NODES = 10000
E = 160000
NSLAB = 2
ESLAB = E // NSLAB
H = 1024
NCH = 4
CW = 128
TE = 1600
TN = 1000

NRANGE = 32
RANGE = 320
N_PAD = NRANGE * RANGE
SUBN = 10
NSUB = RANGE // SUBN
CAP = 3072
GCAP = 160
GP = 80
DSTCHUNK = 4000


def _edge_mlp(ea, w1, b1, w2, b2):
    def body(ea_ref, w1_ref, b1_ref, w2_ref, b2_ref, o_ref):
        x = ea_ref[...].astype(jnp.bfloat16)
        h = jnp.dot(x, w1_ref[...], preferred_element_type=jnp.float32)
        h = jnp.maximum(h + b1_ref[...], 0.0).astype(jnp.bfloat16)
        v = jnp.dot(h, w2_ref[...], preferred_element_type=jnp.float32)
        v = (v + b2_ref[...]).astype(jnp.bfloat16)
        lo = lax.bitcast_convert_type(v[:, :H // 2], jnp.uint16)
        hi = lax.bitcast_convert_type(v[:, H // 2:], jnp.uint16)
        packed = lo.astype(jnp.uint32) | (hi.astype(jnp.uint32) << 16)
        packed = lax.bitcast_convert_type(packed, jnp.int32)
        for c in range(NCH):
            o_ref[c] = packed[:, c * CW:(c + 1) * CW]

    return pl.pallas_call(
        body,
        grid=(ESLAB // TE,),
        in_specs=[
            pl.BlockSpec((TE, 16), lambda i: (i, 0)),
            pl.BlockSpec((16, H), lambda i: (0, 0)),
            pl.BlockSpec((1, H), lambda i: (0, 0)),
            pl.BlockSpec((H, H), lambda i: (0, 0)),
            pl.BlockSpec((1, H), lambda i: (0, 0)),
        ],
        out_specs=pl.BlockSpec((NCH, TE, CW), lambda i: (0, i, 0)),
        out_shape=jax.ShapeDtypeStruct((NCH, ESLAB, CW), jnp.int32),
    )(ea, w1, b1, w2, b2)


def _sc_segment_max(vecflat, dst):
    mesh = plsc.VectorSubcoreMesh(core_axis_name="c", subcore_axis_name="s")

    @functools.partial(
        pl.kernel,
        out_type=jax.ShapeDtypeStruct((N_PAD, NCH * CW), jnp.int32),
        mesh=mesh,
        compiler_params=pltpu.CompilerParams(needs_layout_passes=False),
        scratch_types=[
            pltpu.VMEM((DSTCHUNK,), jnp.int32),
            pltpu.VMEM((CAP + 16,), jnp.int32),
            pltpu.VMEM((CAP + 16,), jnp.int32),
            pltpu.VMEM((CAP + GCAP + 128,), jnp.int32),
            pltpu.VMEM((GCAP,), jnp.int32),
            pltpu.VMEM((GCAP,), jnp.int32),
            pltpu.VMEM((RANGE + 16,), jnp.int32),
            pltpu.VMEM((RANGE + 16,), jnp.int32),
            pltpu.VMEM((RANGE + 16,), jnp.int32),
            pltpu.VMEM((RANGE, CW), jnp.int32),
            pltpu.VMEM((GCAP, CW), jnp.int32),
            pltpu.VMEM((GCAP, CW), jnp.int32),
            pltpu.SemaphoreType.DMA,
            pltpu.SemaphoreType.DMA,
        ],
    )
    def k(vec_hbm, dst_hbm, out_hbm, dstchunk, idbuf, lnbuf, idbuf2,
          idxga, idxgb, counts, offs, cur, acc, rowbufa, rowbufb,
          sema, semb):
        wid = lax.axis_index("s") * 2 + lax.axis_index("c")
        lo = wid * RANGE
        hi = lo + RANGE
        iota = lax.iota(jnp.int32, 16)
        zi16 = jnp.zeros((16,), jnp.int32)
        lane0 = iota == 0

        def sget(ref, i):
            return ref[pl.ds(i, 16)][0]

        def sput(ref, i, v):
            plsc.store_scatter(ref, [jnp.full((16,), i, jnp.int32)],
                               jnp.full((16,), v, jnp.int32), mask=lane0)

        def z1(t, _):
            idbuf2[pl.ds(16 * t, 16)] = zi16
            return 0
        lax.fori_loop(0, (CAP + GCAP + 128) // 16, z1, 0)

        def z2(t, _):
            counts[pl.ds(16 * t, 16)] = zi16
            return 0
        lax.fori_loop(0, RANGE // 16, z2, 0)

        def fchunk(kc, off):
            pltpu.sync_copy(dst_hbm.at[pl.ds(kc * DSTCHUNK, DSTCHUNK)],
                            dstchunk)

            def fvec(t, off):
                d = dstchunk[pl.ds(16 * t, 16)]
                ids = kc * DSTCHUNK + 16 * t + iota
                m = (d >= lo) & (d < hi)
                csum = plsc.cumsum(jnp.where(m, 1, 0))
                posn = off + csum - 1
                plsc.store_scatter(lnbuf, [posn], d - lo, mask=m)
                plsc.store_scatter(idbuf, [posn], ids, mask=m)
                return jnp.minimum(off + csum[15], CAP)

            return lax.fori_loop(0, DSTCHUNK // 16, fvec, off)

        off = lax.fori_loop(0, ESLAB // DSTCHUNK, fchunk, jnp.int32(0))
        nvec = (off + 15) // 16

        ones16 = jnp.ones((16,), jnp.int32)

        def hist(t, _):
            ln = lnbuf[pl.ds(16 * t, 16)]
            m = (16 * t + iota) < off
            plsc.addupdate_scatter(counts, [ln], ones16, mask=m)
            return 0
        lax.fori_loop(0, nvec, hist, 0)

        def pfx(t, carry):
            v = counts[pl.ds(16 * t, 16)]
            inc = plsc.cumsum(v)
            exc = inc - v + carry
            offs[pl.ds(16 * t, 16)] = exc
            cur[pl.ds(16 * t, 16)] = exc
            return carry + jnp.max(inc)
        lax.fori_loop(0, RANGE // 16, pfx, jnp.int32(0))
        sput(offs, jnp.int32(RANGE), off)

        SENT = jnp.int32(0x7FFFFFF)

        def scat(t, _):
            base = 16 * t
            valid = (base + iota) < off
            lns = jnp.where(valid, lnbuf[pl.ds(base, 16)], SENT)
            idv = idbuf[pl.ds(base, 16)]
            sk, sv = plsc.sort_key_val(lns, iota)
            prev = sk.at[jnp.maximum(iota - 1, 0)].get(
                mode="promise_in_bounds")
            runstart = (sk != prev) | (iota == 0)
            firstpos = plsc.cummax(jnp.where(runstart, iota, 0))
            rank = iota - firstpos
            skc = jnp.minimum(sk, RANGE - 1)
            pos = plsc.load_gather(cur, [skc]) + rank
            ids_sorted = idv.at[sv].get(mode="promise_in_bounds")
            validm = sk != SENT
            plsc.store_scatter(idbuf2, [pos], ids_sorted, mask=validm)
            plsc.addupdate_scatter(cur, [skc], ones16, mask=validm)
            return 0
        lax.fori_loop(0, nvec, scat, 0)

        NEG = jnp.full((32,), -jnp.inf, jnp.bfloat16)

        def npieces(s):
            g0 = sget(offs, s * SUBN)
            nrows = sget(offs, s * SUBN + SUBN) - g0
            return g0, (jnp.minimum(nrows, GCAP) + GP - 1) // GP

        def chunk_body(c, _):
            coff = c * ESLAB

            def issue(s, idxg, rowbuf, sem):
                g0, npc = npieces(s)

                def bidx(t, _):
                    idxg[pl.ds(16 * t, 16)] = (
                        idbuf2[pl.ds(g0 + 16 * t, 16)] + coff)
                    return 0
                lax.fori_loop(0, GCAP // 16, bidx, 0)

                def ip(p_, _):
                    pltpu.async_copy(
                        vec_hbm.at[idxg.at[pl.ds(GP * p_, GP)]],
                        rowbuf.at[pl.ds(GP * p_, GP), :], sem)
                    return 0
                lax.fori_loop(0, npc, ip, 0)

            def drain(s, rowbuf, sem):
                _, npc = npieces(s)

                def dp(p_, _):
                    pltpu.make_async_copy(
                        vec_hbm.at[pl.ds(0, GP), :],
                        rowbuf.at[pl.ds(0, GP), :], sem).wait()
                    return 0
                lax.fori_loop(0, npc, dp, 0)

            def reduce(s, rowbuf):
                g0 = sget(offs, s * SUBN)

                def node_body(jn, _):
                    ln = s * SUBN + jn
                    cnt = sget(counts, ln)
                    p = sget(offs, ln) - g0
                    pc = jnp.minimum(p, GCAP - 1)
                    ec = jnp.minimum(p + cnt, GCAP)

                    def rmax(j, ms):
                        return tuple(
                            jnp.maximum(
                                ms[q],
                                plsc.bitcast(rowbuf[j, pl.ds(16 * q, 16)],
                                             jnp.bfloat16))
                            for q in range(CW // 16))
                    ms = lax.fori_loop(pc, ec, rmax,
                                       tuple(NEG for _ in range(CW // 16)))
                    for q in range(CW // 16):
                        acc[ln, pl.ds(16 * q, 16)] = plsc.bitcast(
                            ms[q], jnp.int32)
                    return 0
                lax.fori_loop(0, SUBN, node_body, 0)

            issue(0, idxga, rowbufa, sema)

            def pair_body(sp, _):
                s0 = 2 * sp
                issue(s0 + 1, idxgb, rowbufb, semb)
                drain(s0, rowbufa, sema)
                reduce(s0, rowbufa)

                @pl.when(s0 + 2 < NSUB)
                def _():
                    issue(s0 + 2, idxga, rowbufa, sema)
                drain(s0 + 1, rowbufb, semb)
                reduce(s0 + 1, rowbufb)
                return 0
            lax.fori_loop(0, NSUB // 2, pair_body, 0)

            pltpu.sync_copy(acc,
                            out_hbm.at[pl.ds(lo, RANGE), pl.ds(c * CW, CW)])
            return 0
        lax.fori_loop(0, NCH, chunk_body, 0)

    return k(vecflat, dst)


def _node_mlps(nvs, action, wa1, ba1, wa2, ba2, wf1, bf1, wf2row):
    def body(nva_ref, nvb_ref, act_ref, wa1_ref, ba1_ref, wa2_ref, ba2_ref,
             wf1_ref, bf1_ref, wf2_ref, o_ref):
        a = act_ref[...].astype(jnp.bfloat16)
        t = jnp.dot(a, wa1_ref[...], preferred_element_type=jnp.float32)
        t = jnp.maximum(t + ba1_ref[...], 0.0).astype(jnp.bfloat16)
        emb = jnp.dot(t, wa2_ref[...], preferred_element_type=jnp.float32)
        emb = emb + ba2_ref[...]

        def unpack(ref):
            u = lax.bitcast_convert_type(ref[...], jnp.uint32)
            nvlo = lax.bitcast_convert_type(
                (u & 0xFFFF).astype(jnp.uint16), jnp.bfloat16)
            nvhi = lax.bitcast_convert_type(
                (u >> 16).astype(jnp.uint16), jnp.bfloat16)
            return jnp.concatenate([nvlo, nvhi], axis=1)

        nvv = jnp.maximum(unpack(nva_ref), unpack(nvb_ref))
        nvv = jnp.where(nvv == -jnp.inf, jnp.bfloat16(0), nvv)
        feat = (nvv.astype(jnp.float32) + emb).astype(jnp.bfloat16)
        h = jnp.dot(feat, wf1_ref[...], preferred_element_type=jnp.float32)
        h = jnp.maximum(h + bf1_ref[...], 0.0)
        fld = jnp.sum(h * wf2_ref[...].astype(jnp.float32), axis=1)
        o_ref[pl.program_id(0), :] = fld

    return pl.pallas_call(
        body,
        grid=(N_NODES // TN,),
        in_specs=[
            pl.BlockSpec((TN, H // 2), lambda i: (i, 0)),
            pl.BlockSpec((TN, H // 2), lambda i: (i, 0)),
            pl.BlockSpec((TN, 16), lambda i: (i, 0)),
            pl.BlockSpec((16, H), lambda i: (0, 0)),
            pl.BlockSpec((1, H), lambda i: (0, 0)),
            pl.BlockSpec((H, H), lambda i: (0, 0)),
            pl.BlockSpec((1, H), lambda i: (0, 0)),
            pl.BlockSpec((H, H), lambda i: (0, 0)),
            pl.BlockSpec((1, H), lambda i: (0, 0)),
            pl.BlockSpec((1, H), lambda i: (0, 0)),
        ],
        out_specs=pl.BlockSpec((N_NODES // TN, TN), lambda i: (0, 0)),
        out_shape=jax.ShapeDtypeStruct((N_NODES // TN, TN), jnp.float32),
    )(*nvs, action, wa1, ba1, wa2, ba2, wf1, bf1, wf2row)


def kernel(edge_attr, edge_index, action, Wv1, bv1, Wv2, bv2, Wa1, ba1,
           Wa2, ba2, Wf1, bf1, Wf2, bf2):
    dst = edge_index[1].astype(jnp.int32)
    bf16 = jnp.bfloat16
    w1, b1 = Wv1.astype(bf16), bv1.reshape(1, H)
    w2, b2 = Wv2.astype(bf16), bv2.reshape(1, H)
    nvs = []
    for sl in range(NSLAB):
        vecc = _edge_mlp(edge_attr[sl * ESLAB:(sl + 1) * ESLAB], w1, b1,
                         w2, b2)
        nvs.append(_sc_segment_max(
            vecc.reshape(NCH * ESLAB, CW),
            dst[sl * ESLAB:(sl + 1) * ESLAB]))
    fld = _node_mlps(nvs, action, Wa1.astype(bf16),
                     ba1.reshape(1, H), Wa2.astype(bf16), ba2.reshape(1, H),
                     Wf1.astype(bf16), bf1.reshape(1, H),
                     Wf2.reshape(1, H).astype(bf16))
    return fld.reshape(N_NODES) + bf2[0]


# speedup vs baseline: 4.9990x; 1.0649x over previous
import functools

import jax
import jax.numpy as jnp
from jax import lax
from jax.experimental import pallas as pl
from jax.experimental.pallas import tpu as pltpu
from jax.experimental.pallas import tpu_sc as plsc

N_NODES = 10000
E = 160000
NSLAB = 2
ESLAB = E // NSLAB
H = 1024
NCH = 4
CW = 128
TE = 1600
TN = 1000

NRANGE = 32
RANGE = 320
N_PAD = NRANGE * RANGE
SUBN = 10
NSUB = RANGE // SUBN
CAP = 3072
GCAP = 160
GP = 80
DSTCHUNK = 4000


def _edge_mlp(ea, w1, b1, w2, b2):
    def body(ea_ref, w1_ref, b1_ref, w2_ref, b2_ref, o_ref):
        x = ea_ref[...].astype(jnp.bfloat16)
        h = jnp.dot(x, w1_ref[...], preferred_element_type=jnp.float32)
        h = jnp.maximum(h + b1_ref[...], 0.0).astype(jnp.bfloat16)
        v = jnp.dot(h, w2_ref[...], preferred_element_type=jnp.float32)
        v = (v + b2_ref[...]).astype(jnp.bfloat16)
        lo = lax.bitcast_convert_type(v[:, :H // 2], jnp.uint16)
        hi = lax.bitcast_convert_type(v[:, H // 2:], jnp.uint16)
        packed = lo.astype(jnp.uint32) | (hi.astype(jnp.uint32) << 16)
        packed = lax.bitcast_convert_type(packed, jnp.int32)
        for c in range(NCH):
            o_ref[c] = packed[:, c * CW:(c + 1) * CW]

    return pl.pallas_call(
        body,
        grid=(ESLAB // TE,),
        in_specs=[
            pl.BlockSpec((TE, 16), lambda i: (i, 0)),
            pl.BlockSpec((16, H), lambda i: (0, 0)),
            pl.BlockSpec((1, H), lambda i: (0, 0)),
            pl.BlockSpec((H, H), lambda i: (0, 0)),
            pl.BlockSpec((1, H), lambda i: (0, 0)),
        ],
        out_specs=pl.BlockSpec((NCH, TE, CW), lambda i: (0, i, 0)),
        out_shape=jax.ShapeDtypeStruct((NCH, ESLAB, CW), jnp.int32),
    )(ea, w1, b1, w2, b2)


def _sc_segment_max(vecflat, dst):
    mesh = plsc.VectorSubcoreMesh(core_axis_name="c", subcore_axis_name="s")

    @functools.partial(
        pl.kernel,
        out_type=jax.ShapeDtypeStruct((N_PAD, NCH * CW), jnp.int32),
        mesh=mesh,
        compiler_params=pltpu.CompilerParams(needs_layout_passes=False),
        scratch_types=[
            pltpu.VMEM((DSTCHUNK,), jnp.int32),
            pltpu.VMEM((CAP + 16,), jnp.int32),
            pltpu.VMEM((CAP + 16,), jnp.int32),
            pltpu.VMEM((CAP + GCAP + 128,), jnp.int32),
            pltpu.VMEM((GCAP,), jnp.int32),
            pltpu.VMEM((GCAP,), jnp.int32),
            pltpu.VMEM((RANGE + 16,), jnp.int32),
            pltpu.VMEM((RANGE + 16,), jnp.int32),
            pltpu.VMEM((RANGE + 16,), jnp.int32),
            pltpu.VMEM((RANGE, CW), jnp.int32),
            pltpu.VMEM((GCAP, CW), jnp.int32),
            pltpu.VMEM((GCAP, CW), jnp.int32),
            pltpu.SemaphoreType.DMA,
            pltpu.SemaphoreType.DMA,
        ],
    )
    def k(vec_hbm, dst_hbm, out_hbm, dstchunk, idbuf, lnbuf, idbuf2,
          idxga, idxgb, counts, offs, cur, acc, rowbufa, rowbufb,
          sema, semb):
        wid = lax.axis_index("s") * 2 + lax.axis_index("c")
        lo = wid * RANGE
        hi = lo + RANGE
        iota = lax.iota(jnp.int32, 16)
        zi16 = jnp.zeros((16,), jnp.int32)
        lane0 = iota == 0

        def sget(ref, i):
            return ref[pl.ds(i, 16)][0]

        def sput(ref, i, v):
            plsc.store_scatter(ref, [jnp.full((16,), i, jnp.int32)],
                               jnp.full((16,), v, jnp.int32), mask=lane0)

        def z1(t, _):
            idbuf2[pl.ds(16 * t, 16)] = zi16
            return 0
        lax.fori_loop(0, (CAP + GCAP + 128) // 16, z1, 0)

        def z2(t, _):
            counts[pl.ds(16 * t, 16)] = zi16
            return 0
        lax.fori_loop(0, RANGE // 16, z2, 0)

        def fchunk(kc, off):
            pltpu.sync_copy(dst_hbm.at[pl.ds(kc * DSTCHUNK, DSTCHUNK)],
                            dstchunk)

            def fvec(t, off):
                d = dstchunk[pl.ds(16 * t, 16)]
                ids = kc * DSTCHUNK + 16 * t + iota
                m = (d >= lo) & (d < hi)
                csum = plsc.cumsum(jnp.where(m, 1, 0))
                posn = off + csum - 1
                plsc.store_scatter(lnbuf, [posn], d - lo, mask=m)
                plsc.store_scatter(idbuf, [posn], ids, mask=m)
                return jnp.minimum(off + csum[15], CAP)

            return plsc.parallel_loop(0, DSTCHUNK // 16, unroll=2,
                                      carry=off)(fvec)

        off = lax.fori_loop(0, ESLAB // DSTCHUNK, fchunk, jnp.int32(0))
        nvec = (off + 15) // 16

        ones16 = jnp.ones((16,), jnp.int32)

        def hist(t, _):
            ln = lnbuf[pl.ds(16 * t, 16)]
            m = (16 * t + iota) < off
            plsc.addupdate_scatter(counts, [ln], ones16, mask=m)
            return 0
        lax.fori_loop(0, nvec, hist, 0)

        def pfx(t, carry):
            v = counts[pl.ds(16 * t, 16)]
            inc = plsc.cumsum(v)
            exc = inc - v + carry
            offs[pl.ds(16 * t, 16)] = exc
            cur[pl.ds(16 * t, 16)] = exc
            return carry + jnp.max(inc)
        lax.fori_loop(0, RANGE // 16, pfx, jnp.int32(0))
        sput(offs, jnp.int32(RANGE), off)

        SENT = jnp.int32(0x7FFFFFF)

        def scat(t, _):
            base = 16 * t
            valid = (base + iota) < off
            lns = jnp.where(valid, lnbuf[pl.ds(base, 16)], SENT)
            idv = idbuf[pl.ds(base, 16)]
            sk, sv = plsc.sort_key_val(lns, iota)
            prev = sk.at[jnp.maximum(iota - 1, 0)].get(
                mode="promise_in_bounds")
            runstart = (sk != prev) | (iota == 0)
            firstpos = plsc.cummax(jnp.where(runstart, iota, 0))
            rank = iota - firstpos
            skc = jnp.minimum(sk, RANGE - 1)
            pos = plsc.load_gather(cur, [skc]) + rank
            ids_sorted = idv.at[sv].get(mode="promise_in_bounds")
            validm = sk != SENT
            plsc.store_scatter(idbuf2, [pos], ids_sorted, mask=validm)
            plsc.addupdate_scatter(cur, [skc], ones16, mask=validm)
            return 0
        lax.fori_loop(0, nvec, scat, 0)

        NEG = jnp.full((32,), -jnp.inf, jnp.bfloat16)

        def npieces(s):
            g0 = sget(offs, s * SUBN)
            nrows = sget(offs, s * SUBN + SUBN) - g0
            return g0, (jnp.minimum(nrows, GCAP) + GP - 1) // GP

        def chunk_body(c, _):
            coff = c * ESLAB

            def issue(s, idxg, rowbuf, sem):
                g0, npc = npieces(s)

                def bidx(t, _):
                    idxg[pl.ds(16 * t, 16)] = (
                        idbuf2[pl.ds(g0 + 16 * t, 16)] + coff)
                    return 0
                lax.fori_loop(0, GCAP // 16, bidx, 0)

                def ip(p_, _):
                    pltpu.async_copy(
                        vec_hbm.at[idxg.at[pl.ds(GP * p_, GP)]],
                        rowbuf.at[pl.ds(GP * p_, GP), :], sem)
                    return 0
                lax.fori_loop(0, npc, ip, 0)

            def drain(s, rowbuf, sem):
                _, npc = npieces(s)

                def dp(p_, _):
                    pltpu.make_async_copy(
                        vec_hbm.at[pl.ds(0, GP), :],
                        rowbuf.at[pl.ds(0, GP), :], sem).wait()
                    return 0
                lax.fori_loop(0, npc, dp, 0)

            def reduce(s, rowbuf):
                g0 = sget(offs, s * SUBN)

                def node_body(jn, _):
                    ln = s * SUBN + jn
                    cnt = sget(counts, ln)
                    p = sget(offs, ln) - g0
                    pc = jnp.minimum(p, GCAP - 1)
                    ec = jnp.minimum(p + cnt, GCAP)

                    def rmax(j, ms):
                        return tuple(
                            jnp.maximum(
                                ms[q],
                                plsc.bitcast(rowbuf[j, pl.ds(16 * q, 16)],
                                             jnp.bfloat16))
                            for q in range(CW // 16))
                    ms = plsc.parallel_loop(
                        pc, ec, unroll=2,
                        carry=tuple(NEG for _ in range(CW // 16)))(rmax)
                    for q in range(CW // 16):
                        acc[ln, pl.ds(16 * q, 16)] = plsc.bitcast(
                            ms[q], jnp.int32)
                    return 0
                lax.fori_loop(0, SUBN, node_body, 0)

            issue(0, idxga, rowbufa, sema)

            def pair_body(sp, _):
                s0 = 2 * sp
                issue(s0 + 1, idxgb, rowbufb, semb)
                drain(s0, rowbufa, sema)
                reduce(s0, rowbufa)

                @pl.when(s0 + 2 < NSUB)
                def _():
                    issue(s0 + 2, idxga, rowbufa, sema)
                drain(s0 + 1, rowbufb, semb)
                reduce(s0 + 1, rowbufb)
                return 0
            lax.fori_loop(0, NSUB // 2, pair_body, 0)

            pltpu.sync_copy(acc,
                            out_hbm.at[pl.ds(lo, RANGE), pl.ds(c * CW, CW)])
            return 0
        lax.fori_loop(0, NCH, chunk_body, 0)

    return k(vecflat, dst)


def _node_mlps(nvs, action, wa1, ba1, wa2, ba2, wf1, bf1, wf2row):
    def body(nva_ref, nvb_ref, act_ref, wa1_ref, ba1_ref, wa2_ref, ba2_ref,
             wf1_ref, bf1_ref, wf2_ref, o_ref):
        a = act_ref[...].astype(jnp.bfloat16)
        t = jnp.dot(a, wa1_ref[...], preferred_element_type=jnp.float32)
        t = jnp.maximum(t + ba1_ref[...], 0.0).astype(jnp.bfloat16)
        emb = jnp.dot(t, wa2_ref[...], preferred_element_type=jnp.float32)
        emb = emb + ba2_ref[...]

        def unpack(ref):
            u = lax.bitcast_convert_type(ref[...], jnp.uint32)
            nvlo = lax.bitcast_convert_type(
                (u & 0xFFFF).astype(jnp.uint16), jnp.bfloat16)
            nvhi = lax.bitcast_convert_type(
                (u >> 16).astype(jnp.uint16), jnp.bfloat16)
            return jnp.concatenate([nvlo, nvhi], axis=1)

        nvv = jnp.maximum(unpack(nva_ref), unpack(nvb_ref))
        nvv = jnp.where(nvv == -jnp.inf, jnp.bfloat16(0), nvv)
        feat = (nvv.astype(jnp.float32) + emb).astype(jnp.bfloat16)
        h = jnp.dot(feat, wf1_ref[...], preferred_element_type=jnp.float32)
        h = jnp.maximum(h + bf1_ref[...], 0.0)
        fld = jnp.sum(h * wf2_ref[...].astype(jnp.float32), axis=1)
        o_ref[pl.program_id(0), :] = fld

    return pl.pallas_call(
        body,
        grid=(N_NODES // TN,),
        in_specs=[
            pl.BlockSpec((TN, H // 2), lambda i: (i, 0)),
            pl.BlockSpec((TN, H // 2), lambda i: (i, 0)),
            pl.BlockSpec((TN, 16), lambda i: (i, 0)),
            pl.BlockSpec((16, H), lambda i: (0, 0)),
            pl.BlockSpec((1, H), lambda i: (0, 0)),
            pl.BlockSpec((H, H), lambda i: (0, 0)),
            pl.BlockSpec((1, H), lambda i: (0, 0)),
            pl.BlockSpec((H, H), lambda i: (0, 0)),
            pl.BlockSpec((1, H), lambda i: (0, 0)),
            pl.BlockSpec((1, H), lambda i: (0, 0)),
        ],
        out_specs=pl.BlockSpec((N_NODES // TN, TN), lambda i: (0, 0)),
        out_shape=jax.ShapeDtypeStruct((N_NODES // TN, TN), jnp.float32),
    )(*nvs, action, wa1, ba1, wa2, ba2, wf1, bf1, wf2row)


def kernel(edge_attr, edge_index, action, Wv1, bv1, Wv2, bv2, Wa1, ba1,
           Wa2, ba2, Wf1, bf1, Wf2, bf2):
    dst = edge_index[1].astype(jnp.int32)
    bf16 = jnp.bfloat16
    w1, b1 = Wv1.astype(bf16), bv1.reshape(1, H)
    w2, b2 = Wv2.astype(bf16), bv2.reshape(1, H)
    nvs = []
    for sl in range(NSLAB):
        vecc = _edge_mlp(edge_attr[sl * ESLAB:(sl + 1) * ESLAB], w1, b1,
                         w2, b2)
        nvs.append(_sc_segment_max(
            vecc.reshape(NCH * ESLAB, CW),
            dst[sl * ESLAB:(sl + 1) * ESLAB]))
    fld = _node_mlps(nvs, action, Wa1.astype(bf16),
                     ba1.reshape(1, H), Wa2.astype(bf16), ba2.reshape(1, H),
                     Wf1.astype(bf16), bf1.reshape(1, H),
                     Wf2.reshape(1, H).astype(bf16))
    return fld.reshape(N_NODES) + bf2[0]
